# Initial kernel scaffold; baseline (speedup 1.0000x reference)
import jax
import jax.numpy as jnp
from jax.experimental import pallas as pl


def kernel(x, edge_index, W, b):
    raise NotImplementedError("write your pallas kernel here")


# speedup vs baseline: 20.0024x; 20.0024x over previous
import functools

import jax
import jax.numpy as jnp
from jax import lax
from jax.experimental import pallas as pl
from jax.experimental.pallas import tpu as pltpu
from jax.experimental.pallas import tpu_sc as plsc

CH = 128
N_PAD = 10240
NW = 32
C = 128
L = 16
BLK = 256


def _sc_mesh():
    return plsc.VectorSubcoreMesh(core_axis_name="c", subcore_axis_name="s")


@functools.lru_cache(maxsize=None)
def _deg_call(nch):
    @functools.partial(
        pl.kernel,
        out_type=jax.ShapeDtypeStruct((NW, N_PAD), jnp.float32),
        mesh=_sc_mesh(),
        scratch_types=[
            pltpu.VMEM((nch, C), jnp.int32),
            pltpu.VMEM((nch, C), jnp.float32),
            pltpu.VMEM((N_PAD,), jnp.float32),
        ],
        compiler_params=pltpu.CompilerParams(needs_layout_passes=False),
    )
    def deg_kernel(dst_hbm, w_hbm, degp_hbm, didx, wbuf, hist):
        cid = lax.axis_index("c")
        sid = lax.axis_index("s")
        wid = sid * 2 + cid
        pltpu.sync_copy(dst_hbm.at[wid], didx)
        pltpu.sync_copy(w_hbm.at[wid], wbuf)
        zeros = jnp.zeros((L,), jnp.float32)

        def zero_body(i, carry):
            hist[pl.ds(i * L, L)] = zeros
            return carry

        lax.fori_loop(0, N_PAD // L, zero_body, 0)

        def vec_body(j, carry):
            g = j // (C // L)
            k = j % (C // L)
            idx16 = didx[g, pl.ds(k * L, L)]
            w16 = wbuf[g, pl.ds(k * L, L)]
            plsc.addupdate_scatter(hist, [idx16], w16)
            return carry

        lax.fori_loop(0, nch * (C // L), vec_body, 0)
        pltpu.sync_copy(hist, degp_hbm.at[wid])

    return deg_kernel


@functools.lru_cache(maxsize=None)
def _scatter_call(nch):
    @functools.partial(
        pl.kernel,
        out_type=jax.ShapeDtypeStruct((2, N_PAD, CH), jnp.float32),
        mesh=_sc_mesh(),
        scratch_types=[
            pltpu.VMEM((nch, C), jnp.int32),
            pltpu.VMEM((nch, C), jnp.int32),
            pltpu.VMEM((C, CH), jnp.float32),
            pltpu.VMEM_SHARED((N_PAD, CH), jnp.float32),
            pltpu.SemaphoreType.DMA,
        ],
        compiler_params=pltpu.CompilerParams(needs_layout_passes=False),
    )
    def scatter_kernel(h2_hbm, src_hbm, dst_hbm, out_hbm, sidx, didx, rows,
                       accum, sem):
        cid = lax.axis_index("c")
        sid = lax.axis_index("s")
        wid = sid * 2 + cid
        pltpu.sync_copy(src_hbm.at[wid], sidx)
        pltpu.sync_copy(dst_hbm.at[wid], didx)
        zeros = jnp.zeros((L,), jnp.float32)

        def zrow(i, carry):
            r = i // (CH // L)
            k = i % (CH // L)
            rows[r, pl.ds(k * L, L)] = zeros
            return carry

        lax.fori_loop(0, C * (CH // L), zrow, 0)
        rows_per_tile = N_PAD // 16
        base = sid * rows_per_tile

        def zslice(k, carry):
            pltpu.sync_copy(rows, accum.at[pl.ds(base + k * C, C)])
            return carry

        lax.fori_loop(0, rows_per_tile // C, zslice, 0)
        plsc.subcore_barrier()

        def chunk(g, carry):
            pltpu.async_copy(h2_hbm.at[sidx.at[g]], rows, sem).wait()
            pltpu.sync_copy(rows, accum.at[didx.at[g]], add=True)
            return carry

        lax.fori_loop(0, nch, chunk, 0)
        plsc.subcore_barrier()
        pltpu.sync_copy(accum.at[pl.ds(base, rows_per_tile)],
                        out_hbm.at[cid, pl.ds(base, rows_per_tile)])

    return scatter_kernel


def _h2_body(x_ref, w_ref, degp_ref, h2_ref):
    deg = jnp.sum(degp_ref[...], axis=0)
    dis = lax.rsqrt(jnp.maximum(deg, 1.0))
    h = jnp.dot(x_ref[...], w_ref[...], preferred_element_type=jnp.float32)
    h2_ref[...] = h * dis[:, None]


_h2_kernel = pl.pallas_call(
    _h2_body,
    grid=(N_PAD // BLK,),
    in_specs=[
        pl.BlockSpec((BLK, CH), lambda i: (i, 0)),
        pl.BlockSpec((CH, CH), lambda i: (0, 0)),
        pl.BlockSpec((NW, BLK), lambda i: (0, i)),
    ],
    out_specs=pl.BlockSpec((BLK, CH), lambda i: (i, 0)),
    out_shape=jax.ShapeDtypeStruct((N_PAD, CH), jnp.float32),
)


def _out_body(p0_ref, p1_ref, degp_ref, b_ref, o_ref):
    deg = jnp.sum(degp_ref[...], axis=0)
    dis = lax.rsqrt(jnp.maximum(deg, 1.0))
    acc = (p0_ref[...] + p1_ref[...]) * dis[:, None]
    o_ref[...] = jnp.maximum(acc + b_ref[...], 0.0)


_out_kernel = pl.pallas_call(
    _out_body,
    grid=(N_PAD // BLK,),
    in_specs=[
        pl.BlockSpec((BLK, CH), lambda i: (i, 0)),
        pl.BlockSpec((BLK, CH), lambda i: (i, 0)),
        pl.BlockSpec((NW, BLK), lambda i: (0, i)),
        pl.BlockSpec((1, CH), lambda i: (0, 0)),
    ],
    out_specs=pl.BlockSpec((BLK, CH), lambda i: (i, 0)),
    out_shape=jax.ShapeDtypeStruct((N_PAD, CH), jnp.float32),
)


@jax.jit
def kernel(x, edge_index, W, b):
    n = x.shape[0]
    src = edge_index[0].astype(jnp.int32)
    dst = edge_index[1].astype(jnp.int32)
    loop = jnp.arange(n, dtype=jnp.int32)
    e_real = src.shape[0] + n
    ept = -(-e_real // (NW * C)) * C
    e_pad = ept * NW
    nch = ept // C
    pad = e_pad - e_real
    fill = jnp.full((pad,), n, jnp.int32)
    src_all = jnp.concatenate([src, loop, fill]).reshape(NW, nch, C)
    dst_all = jnp.concatenate([dst, loop, fill]).reshape(NW, nch, C)
    w_all = jnp.concatenate([
        jnp.ones((e_real,), jnp.float32),
        jnp.zeros((pad,), jnp.float32),
    ]).reshape(NW, nch, C)
    x_pad = jnp.concatenate([x, jnp.zeros((N_PAD - n, CH), x.dtype)], axis=0)

    degp = _deg_call(nch)(dst_all, w_all)
    h2 = _h2_kernel(x_pad, W, degp)
    partials = _scatter_call(nch)(h2, src_all, dst_all)
    out = _out_kernel(partials[0], partials[1], degp, b.reshape(1, CH))
    return out[:n]
